# baseline (speedup 1.0000x reference)
import functools

import jax
import jax.numpy as jnp
from jax import lax
from jax.experimental import pallas as pl
from jax.experimental.pallas import tpu as pltpu
from jax.experimental.pallas import tpu_sc as plsc

K_NB = 128
D = 64
N_KEYS = 1_000_000
N_TILES = 32
LANES = 16

N_T = ((N_KEYS + N_TILES - 1) // N_TILES + 127) // 128 * 128
N_PAD = N_T * N_TILES
NV = N_T // LANES

B1 = 12544 if N_PAD % 12544 == 0 else N_T
G1 = N_PAD // B1


def _dist_kernel(q_ref, k_ref, o_ref):
    qv = q_ref[...]
    kb = k_ref[...]
    dn = (((1,), (1,)), ((), ()))
    kq = lax.dot_general(qv, kb, dn)
    ksq = lax.dot_general(jnp.ones((1, D), jnp.float32), kb * kb, dn,
                          precision=lax.Precision.HIGHEST)
    qsq = jnp.sum(qv * qv)
    d2 = (qsq + ksq) - 2.0 * kq
    d2 = jnp.maximum(d2, 0.0)
    dist = jnp.sqrt(d2 + 1e-12)
    bits = pltpu.bitcast(dist, jnp.int32)
    i = pl.program_id(0)
    gid = i * B1 + lax.broadcasted_iota(jnp.int32, (1, B1), 1)
    o_ref[...] = jnp.where(gid < N_KEYS, bits, jnp.int32(0x7F800000))[None]


_dist_call = pl.pallas_call(
    _dist_kernel,
    grid=(G1,),
    in_specs=[
        pl.BlockSpec((1, D), lambda i: (0, 0)),
        pl.BlockSpec((B1, D), lambda i: (i, 0)),
    ],
    out_specs=pl.BlockSpec((1, 1, B1), lambda i: (i, 0, 0)),
    out_shape=jax.ShapeDtypeStruct((G1, 1, B1), jnp.int32),
)


def _select_body(dists, out_v, out_i, vals, hist, racc, ltv, lti, eqi, ov, oi):
    c = lax.axis_index("c")
    s = lax.axis_index("s")
    wid = s * 2 + c
    base = wid * N_T
    pltpu.sync_copy(dists.at[pl.ds(base, N_T)], vals)

    lane = lax.iota(jnp.int32, LANES)
    ones = jnp.ones((LANES,), jnp.int32)
    zeros16 = jnp.zeros((LANES,), jnp.int32)

    prefix = jnp.int32(0)
    k_rem = jnp.int32(K_NB)
    for r in range(4):
        sh_d = 24 - 8 * r

        def zbody(j, _):
            hist[pl.ds(j * LANES, LANES)] = zeros16
            return 0

        lax.fori_loop(0, 256, zbody, 0)

        if r == 0:
            def hbody(i, _):
                u = vals[pl.ds(i * LANES, LANES)]
                byte = (u >> sh_d) & 255
                plsc.addupdate_scatter(hist, [byte * LANES + lane], ones)
                return 0
        else:
            sh_hi = 32 - 8 * r
            pref_hi = prefix >> sh_hi

            def hbody(i, _):
                u = vals[pl.ds(i * LANES, LANES)]
                byte = (u >> sh_d) & 255
                ok = (u >> sh_hi) == pref_hi
                plsc.addupdate_scatter(hist, [byte * LANES + lane], ones,
                                       mask=ok)
                return 0

        lax.fori_loop(0, NV, hbody, 0)

        def cbody(j, acc):
            acc = acc + hist[pl.ds(j * LANES, LANES)]
            racc[pl.ds(j * LANES, LANES)] = acc
            return acc

        lax.fori_loop(0, 256, cbody, zeros16)

        def sbody(_, lohi):
            lo, hi = lohi
            mid = (lo + hi) // 2
            sm = jnp.sum(racc[pl.ds(mid * LANES, LANES)])
            ok = sm >= k_rem
            return jnp.where(ok, lo, mid + 1), jnp.where(ok, mid, hi)

        digit, _ = lax.fori_loop(0, 8, sbody, (jnp.int32(0), jnp.int32(255)))
        pm1 = jnp.maximum(digit - 1, 0)
        cum_before = jnp.where(
            digit > 0, jnp.sum(racc[pl.ds(pm1 * LANES, LANES)]), jnp.int32(0))
        k_rem = k_rem - cum_before
        prefix = prefix | (digit << sh_d)

    v_bits = prefix

    def pbody(i, ptrs):
        p_lt, p_eq = ptrs
        u = vals[pl.ds(i * LANES, LANES)]
        lt = u < v_bits
        eq = u == v_bits
        gidx = base + i * LANES + lane
        plsc.store_compressed(ltv.at[pl.ds(p_lt, LANES)], u, mask=lt)
        plsc.store_compressed(lti.at[pl.ds(p_lt, LANES)], gidx, mask=lt)
        plsc.store_compressed(eqi.at[pl.ds(p_eq, LANES)], gidx, mask=eq)
        return (p_lt + jnp.sum(lt.astype(jnp.int32)),
                p_eq + jnp.sum(eq.astype(jnp.int32)))

    lax.fori_loop(0, NV, pbody, (jnp.int32(0), jnp.int32(0)))

    count_lt = jnp.int32(K_NB) - k_rem
    vfull = jnp.broadcast_to(v_bits, (LANES,))
    for j in range(K_NB // LANES):
        pos = j * LANES + lane
        sel = pos < count_lt
        lv = ltv[pl.ds(j * LANES, LANES)]
        li = lti[pl.ds(j * LANES, LANES)]
        ei = plsc.load_gather(eqi, [jnp.maximum(pos - count_lt, 0)])
        ov[pl.ds(j * LANES, LANES)] = jnp.where(sel, lv, vfull)
        oi[pl.ds(j * LANES, LANES)] = jnp.where(sel, li, ei)

    pltpu.sync_copy(ov, out_v.at[pl.ds(wid * K_NB, K_NB)])
    pltpu.sync_copy(oi, out_i.at[pl.ds(wid * K_NB, K_NB)])


@functools.lru_cache(maxsize=1)
def _make_select_call():
  return pl.kernel(
    _select_body,
    out_type=(
        jax.ShapeDtypeStruct((N_TILES * K_NB,), jnp.int32),
        jax.ShapeDtypeStruct((N_TILES * K_NB,), jnp.int32),
    ),
    mesh=plsc.VectorSubcoreMesh(
        core_axis_name="c", subcore_axis_name="s",
        num_cores=2, num_subcores=16),
    compiler_params=pltpu.CompilerParams(needs_layout_passes=False),
    scratch_types=[
        pltpu.VMEM((N_T,), jnp.int32),
        pltpu.VMEM((256 * LANES,), jnp.int32),
        pltpu.VMEM((256 * LANES,), jnp.int32),
        pltpu.VMEM((K_NB + LANES,), jnp.int32),
        pltpu.VMEM((K_NB + LANES,), jnp.int32),
        pltpu.VMEM((N_T + LANES,), jnp.int32),
        pltpu.VMEM((K_NB,), jnp.int32),
        pltpu.VMEM((K_NB,), jnp.int32),
    ],
  )


def _merge_kernel(cv_ref, ci_ref, ov_ref, oi_ref):
    lane = lax.broadcasted_iota(jnp.int32, (1, K_NB), 1)
    cand = pltpu.bitcast(cv_ref[...], jnp.float32)

    def body(j, carry):
        vals, oval, oidx = carry
        m = jnp.min(vals)
        sel = vals == m
        i = jnp.min(jnp.where(sel, ci_ref[...], jnp.int32(2**31 - 1)))
        vals = jnp.where(sel & (ci_ref[...] == i), jnp.float32(jnp.inf), vals)
        oval = jnp.where(lane == j, m, oval)
        oidx = jnp.where(lane == j, i, oidx)
        return vals, oval, oidx

    _, oval, oidx = lax.fori_loop(
        0, K_NB, body,
        (cand, jnp.zeros((1, K_NB), jnp.float32),
         jnp.zeros((1, K_NB), jnp.int32)))
    ov_ref[...] = oval
    oi_ref[...] = oidx


_merge_call = pl.pallas_call(
    _merge_kernel,
    out_shape=(
        jax.ShapeDtypeStruct((1, K_NB), jnp.float32),
        jax.ShapeDtypeStruct((1, K_NB), jnp.int32),
    ),
)


def kernel(queries, keys):
    dists = _dist_call(queries, keys).reshape(-1)
    cand_v, cand_i = _make_select_call()(dists)
    return _merge_call(cand_v.reshape(N_TILES, K_NB),
                       cand_i.reshape(N_TILES, K_NB))


# speedup vs baseline: 3.6523x; 3.6523x over previous
import functools

import jax
import jax.numpy as jnp
from jax import lax
from jax.experimental import pallas as pl
from jax.experimental.pallas import tpu as pltpu
from jax.experimental.pallas import tpu_sc as plsc

K_NB = 128
D = 64
N_KEYS = 1_000_000
N_TILES = 32
LANES = 16

N_T = ((N_KEYS + N_TILES - 1) // N_TILES + 127) // 128 * 128
N_PAD = N_T * N_TILES
NV = N_T // LANES

B1 = 12544 if N_PAD % 12544 == 0 else N_T
G1 = N_PAD // B1


def _dist_kernel(q_ref, k_ref, o_ref):
    qv = q_ref[...]
    kb = k_ref[...]
    kq = lax.dot_general(qv, kb, (((1,), (0,)), ((), ())))
    ksq = jnp.sum(kb * kb, axis=0, keepdims=True)
    qsq = jnp.sum(qv * qv)
    d2 = (qsq + ksq) - 2.0 * kq
    d2 = jnp.maximum(d2, 0.0)
    dist = jnp.sqrt(d2 + 1e-12)
    bits = pltpu.bitcast(dist, jnp.int32)
    i = pl.program_id(0)
    gid = i * B1 + lax.broadcasted_iota(jnp.int32, (1, B1), 1)
    o_ref[...] = jnp.where(gid < N_KEYS, bits, jnp.int32(0x7F800000))[None]


_dist_call = pl.pallas_call(
    _dist_kernel,
    grid=(G1,),
    in_specs=[
        pl.BlockSpec((1, D), lambda i: (0, 0)),
        pl.BlockSpec((D, B1), lambda i: (0, i)),
    ],
    out_specs=pl.BlockSpec((1, 1, B1), lambda i: (i, 0, 0)),
    out_shape=jax.ShapeDtypeStruct((G1, 1, B1), jnp.int32),
)


def _select_body(dists, out_v, out_i, vals, hist, racc, ltv, lti, eqi, ov, oi):
    c = lax.axis_index("c")
    s = lax.axis_index("s")
    wid = s * 2 + c
    base = wid * N_T
    pltpu.sync_copy(dists.at[pl.ds(base, N_T)], vals)

    lane = lax.iota(jnp.int32, LANES)
    ones = jnp.ones((LANES,), jnp.int32)
    zeros16 = jnp.zeros((LANES,), jnp.int32)

    prefix = jnp.int32(0)
    k_rem = jnp.int32(K_NB)
    for r in range(4):
        sh_d = 24 - 8 * r

        def zbody(j, _):
            hist[pl.ds(j * LANES, LANES)] = zeros16
            return 0

        lax.fori_loop(0, 256, zbody, 0)

        if r == 0:
            def hbody(i, _):
                u = vals[pl.ds(i * LANES, LANES)]
                byte = (u >> sh_d) & 255
                plsc.addupdate_scatter(hist, [byte * LANES + lane], ones)
                return 0
        else:
            sh_hi = 32 - 8 * r
            pref_hi = prefix >> sh_hi

            def hbody(i, _):
                u = vals[pl.ds(i * LANES, LANES)]
                byte = (u >> sh_d) & 255
                ok = (u >> sh_hi) == pref_hi
                plsc.addupdate_scatter(hist, [byte * LANES + lane], ones,
                                       mask=ok)
                return 0

        lax.fori_loop(0, NV, hbody, 0)

        def cbody(j, acc):
            acc = acc + hist[pl.ds(j * LANES, LANES)]
            racc[pl.ds(j * LANES, LANES)] = acc
            return acc

        lax.fori_loop(0, 256, cbody, zeros16)

        def sbody(_, lohi):
            lo, hi = lohi
            mid = (lo + hi) // 2
            sm = jnp.sum(racc[pl.ds(mid * LANES, LANES)])
            ok = sm >= k_rem
            return jnp.where(ok, lo, mid + 1), jnp.where(ok, mid, hi)

        digit, _ = lax.fori_loop(0, 8, sbody, (jnp.int32(0), jnp.int32(255)))
        pm1 = jnp.maximum(digit - 1, 0)
        cum_before = jnp.where(
            digit > 0, jnp.sum(racc[pl.ds(pm1 * LANES, LANES)]), jnp.int32(0))
        k_rem = k_rem - cum_before
        prefix = prefix | (digit << sh_d)

    v_bits = prefix

    def pbody(i, ptrs):
        p_lt, p_eq = ptrs
        u = vals[pl.ds(i * LANES, LANES)]
        lt = u < v_bits
        eq = u == v_bits
        gidx = base + i * LANES + lane
        plsc.store_compressed(ltv.at[pl.ds(p_lt, LANES)], u, mask=lt)
        plsc.store_compressed(lti.at[pl.ds(p_lt, LANES)], gidx, mask=lt)
        plsc.store_compressed(eqi.at[pl.ds(p_eq, LANES)], gidx, mask=eq)
        return (p_lt + jnp.sum(lt.astype(jnp.int32)),
                p_eq + jnp.sum(eq.astype(jnp.int32)))

    lax.fori_loop(0, NV, pbody, (jnp.int32(0), jnp.int32(0)))

    count_lt = jnp.int32(K_NB) - k_rem
    vfull = jnp.broadcast_to(v_bits, (LANES,))
    for j in range(K_NB // LANES):
        pos = j * LANES + lane
        sel = pos < count_lt
        lv = ltv[pl.ds(j * LANES, LANES)]
        li = lti[pl.ds(j * LANES, LANES)]
        ei = plsc.load_gather(eqi, [jnp.maximum(pos - count_lt, 0)])
        ov[pl.ds(j * LANES, LANES)] = jnp.where(sel, lv, vfull)
        oi[pl.ds(j * LANES, LANES)] = jnp.where(sel, li, ei)

    pltpu.sync_copy(ov, out_v.at[pl.ds(wid * K_NB, K_NB)])
    pltpu.sync_copy(oi, out_i.at[pl.ds(wid * K_NB, K_NB)])


@functools.lru_cache(maxsize=1)
def _make_select_call():
  return pl.kernel(
    _select_body,
    out_type=(
        jax.ShapeDtypeStruct((N_TILES * K_NB,), jnp.int32),
        jax.ShapeDtypeStruct((N_TILES * K_NB,), jnp.int32),
    ),
    mesh=plsc.VectorSubcoreMesh(
        core_axis_name="c", subcore_axis_name="s",
        num_cores=2, num_subcores=16),
    compiler_params=pltpu.CompilerParams(needs_layout_passes=False),
    scratch_types=[
        pltpu.VMEM((N_T,), jnp.int32),
        pltpu.VMEM((256 * LANES,), jnp.int32),
        pltpu.VMEM((256 * LANES,), jnp.int32),
        pltpu.VMEM((K_NB + LANES,), jnp.int32),
        pltpu.VMEM((K_NB + LANES,), jnp.int32),
        pltpu.VMEM((N_T + LANES,), jnp.int32),
        pltpu.VMEM((K_NB,), jnp.int32),
        pltpu.VMEM((K_NB,), jnp.int32),
    ],
  )


def _merge_kernel(cv_ref, ci_ref, ov_ref, oi_ref):
    lane = lax.broadcasted_iota(jnp.int32, (1, K_NB), 1)
    cand = pltpu.bitcast(cv_ref[...], jnp.float32)

    def body(j, carry):
        vals, oval, oidx = carry
        m = jnp.min(vals)
        sel = vals == m
        i = jnp.min(jnp.where(sel, ci_ref[...], jnp.int32(2**31 - 1)))
        vals = jnp.where(sel & (ci_ref[...] == i), jnp.float32(jnp.inf), vals)
        oval = jnp.where(lane == j, m, oval)
        oidx = jnp.where(lane == j, i, oidx)
        return vals, oval, oidx

    _, oval, oidx = lax.fori_loop(
        0, K_NB, body,
        (cand, jnp.zeros((1, K_NB), jnp.float32),
         jnp.zeros((1, K_NB), jnp.int32)))
    ov_ref[...] = oval
    oi_ref[...] = oidx


_merge_call = pl.pallas_call(
    _merge_kernel,
    out_shape=(
        jax.ShapeDtypeStruct((1, K_NB), jnp.float32),
        jax.ShapeDtypeStruct((1, K_NB), jnp.int32),
    ),
)


def kernel(queries, keys):
    dists = _dist_call(queries, keys.T).reshape(-1)
    cand_v, cand_i = _make_select_call()(dists)
    return _merge_call(cand_v.reshape(N_TILES, K_NB),
                       cand_i.reshape(N_TILES, K_NB))


# speedup vs baseline: 3.8309x; 1.0489x over previous
import functools

import jax
import jax.numpy as jnp
from jax import lax
from jax.experimental import pallas as pl
from jax.experimental.pallas import tpu as pltpu
from jax.experimental.pallas import tpu_sc as plsc

K_NB = 128
D = 64
N_KEYS = 1_000_000
N_TILES = 32
LANES = 16

N_T = ((N_KEYS + N_TILES - 1) // N_TILES + 127) // 128 * 128
N_PAD = N_T * N_TILES
NV = N_T // LANES

B1 = 12544 if N_PAD % 12544 == 0 else N_T
G1 = N_PAD // B1


def _dist_kernel(q_ref, k_ref, o_ref):
    qv = q_ref[...]
    kb = k_ref[...]
    kq = lax.dot_general(qv, kb, (((1,), (0,)), ((), ())))
    ksq = jnp.sum(kb * kb, axis=0, keepdims=True)
    qsq = jnp.sum(qv * qv)
    d2 = (qsq + ksq) - 2.0 * kq
    d2 = jnp.maximum(d2, 0.0)
    dist = jnp.sqrt(d2 + 1e-12)
    bits = pltpu.bitcast(dist, jnp.int32)
    i = pl.program_id(0)
    gid = i * B1 + lax.broadcasted_iota(jnp.int32, (1, B1), 1)
    o_ref[...] = jnp.where(gid < N_KEYS, bits, jnp.int32(0x7F800000))[None]


_dist_call = pl.pallas_call(
    _dist_kernel,
    grid=(G1,),
    in_specs=[
        pl.BlockSpec((1, D), lambda i: (0, 0)),
        pl.BlockSpec((D, B1), lambda i: (0, i)),
    ],
    out_specs=pl.BlockSpec((1, 1, B1), lambda i: (i, 0, 0)),
    out_shape=jax.ShapeDtypeStruct((G1, 1, B1), jnp.int32),
)


def _select_body(dists, out_v, out_i, vals, hist, racc, ltv, lti, eqi, ov, oi):
    c = lax.axis_index("c")
    s = lax.axis_index("s")
    wid = s * 2 + c
    base = wid * N_T
    pltpu.sync_copy(dists.at[pl.ds(base, N_T)], vals)

    lane = lax.iota(jnp.int32, LANES)
    ones = jnp.ones((LANES,), jnp.int32)
    zeros16 = jnp.zeros((LANES,), jnp.int32)

    prefix = jnp.int32(0)
    k_rem = jnp.int32(K_NB)
    UN = 8
    for r in range(4):
        sh_d = 24 - 8 * r

        def zbody(jo, _):
            for u in range(4):
                hist[pl.ds((jo * 4 + u) * LANES, LANES)] = zeros16
            return 0

        lax.fori_loop(0, 256 // 4, zbody, 0)

        if r == 0:
            def hbody(io, _):
                for u in range(UN):
                    i = io * UN + u
                    v = vals[pl.ds(i * LANES, LANES)]
                    byte = (v >> sh_d) & 255
                    plsc.addupdate_scatter(hist, [byte * LANES + lane], ones)
                return 0
        else:
            sh_hi = 32 - 8 * r
            pref_hi = prefix >> sh_hi

            def hbody(io, _):
                for u in range(UN):
                    i = io * UN + u
                    v = vals[pl.ds(i * LANES, LANES)]
                    byte = (v >> sh_d) & 255
                    ok = (v >> sh_hi) == pref_hi
                    plsc.addupdate_scatter(hist, [byte * LANES + lane], ones,
                                           mask=ok)
                return 0

        lax.fori_loop(0, NV // UN, hbody, 0)

        def cbody(jo, acc):
            for u in range(4):
                acc = acc + hist[pl.ds((jo * 4 + u) * LANES, LANES)]
                racc[pl.ds((jo * 4 + u) * LANES, LANES)] = acc
            return acc

        lax.fori_loop(0, 256 // 4, cbody, zeros16)

        def sbody(_, lohi):
            lo, hi = lohi
            mid = (lo + hi) // 2
            sm = jnp.sum(racc[pl.ds(mid * LANES, LANES)])
            ok = sm >= k_rem
            return jnp.where(ok, lo, mid + 1), jnp.where(ok, mid, hi)

        digit, _ = lax.fori_loop(0, 8, sbody, (jnp.int32(0), jnp.int32(255)))
        pm1 = jnp.maximum(digit - 1, 0)
        cum_before = jnp.where(
            digit > 0, jnp.sum(racc[pl.ds(pm1 * LANES, LANES)]), jnp.int32(0))
        k_rem = k_rem - cum_before
        prefix = prefix | (digit << sh_d)

    v_bits = prefix

    def pbody(io, ptrs):
        p_lt, p_eq = ptrs
        for u in range(UN):
            i = io * UN + u
            v = vals[pl.ds(i * LANES, LANES)]
            lt = v < v_bits
            eq = v == v_bits
            gidx = base + i * LANES + lane
            plsc.store_compressed(ltv.at[pl.ds(p_lt, LANES)], v, mask=lt)
            plsc.store_compressed(lti.at[pl.ds(p_lt, LANES)], gidx, mask=lt)
            plsc.store_compressed(eqi.at[pl.ds(p_eq, LANES)], gidx, mask=eq)
            p_lt = p_lt + plsc.all_reduce_population_count(lt)[0]
            p_eq = p_eq + plsc.all_reduce_population_count(eq)[0]
        return (p_lt, p_eq)

    lax.fori_loop(0, NV // UN, pbody, (jnp.int32(0), jnp.int32(0)))

    count_lt = jnp.int32(K_NB) - k_rem
    vfull = jnp.broadcast_to(v_bits, (LANES,))
    for j in range(K_NB // LANES):
        pos = j * LANES + lane
        sel = pos < count_lt
        lv = ltv[pl.ds(j * LANES, LANES)]
        li = lti[pl.ds(j * LANES, LANES)]
        ei = plsc.load_gather(eqi, [jnp.maximum(pos - count_lt, 0)])
        ov[pl.ds(j * LANES, LANES)] = jnp.where(sel, lv, vfull)
        oi[pl.ds(j * LANES, LANES)] = jnp.where(sel, li, ei)

    pltpu.sync_copy(ov, out_v.at[pl.ds(wid * K_NB, K_NB)])
    pltpu.sync_copy(oi, out_i.at[pl.ds(wid * K_NB, K_NB)])


@functools.lru_cache(maxsize=1)
def _make_select_call():
  return pl.kernel(
    _select_body,
    out_type=(
        jax.ShapeDtypeStruct((N_TILES * K_NB,), jnp.int32),
        jax.ShapeDtypeStruct((N_TILES * K_NB,), jnp.int32),
    ),
    mesh=plsc.VectorSubcoreMesh(
        core_axis_name="c", subcore_axis_name="s",
        num_cores=2, num_subcores=16),
    compiler_params=pltpu.CompilerParams(needs_layout_passes=False),
    scratch_types=[
        pltpu.VMEM((N_T,), jnp.int32),
        pltpu.VMEM((256 * LANES,), jnp.int32),
        pltpu.VMEM((256 * LANES,), jnp.int32),
        pltpu.VMEM((K_NB + LANES,), jnp.int32),
        pltpu.VMEM((K_NB + LANES,), jnp.int32),
        pltpu.VMEM((N_T + LANES,), jnp.int32),
        pltpu.VMEM((K_NB,), jnp.int32),
        pltpu.VMEM((K_NB,), jnp.int32),
    ],
  )


def _merge_kernel(cv_ref, ci_ref, ov_ref, oi_ref):
    lane = lax.broadcasted_iota(jnp.int32, (1, K_NB), 1)
    cand = pltpu.bitcast(cv_ref[...], jnp.float32)

    def body(j, carry):
        vals, oval, oidx = carry
        m = jnp.min(vals)
        sel = vals == m
        i = jnp.min(jnp.where(sel, ci_ref[...], jnp.int32(2**31 - 1)))
        vals = jnp.where(sel & (ci_ref[...] == i), jnp.float32(jnp.inf), vals)
        oval = jnp.where(lane == j, m, oval)
        oidx = jnp.where(lane == j, i, oidx)
        return vals, oval, oidx

    _, oval, oidx = lax.fori_loop(
        0, K_NB, body,
        (cand, jnp.zeros((1, K_NB), jnp.float32),
         jnp.zeros((1, K_NB), jnp.int32)))
    ov_ref[...] = oval
    oi_ref[...] = oidx


_merge_call = pl.pallas_call(
    _merge_kernel,
    out_shape=(
        jax.ShapeDtypeStruct((1, K_NB), jnp.float32),
        jax.ShapeDtypeStruct((1, K_NB), jnp.int32),
    ),
)


def kernel(queries, keys):
    dists = _dist_call(queries, keys.T).reshape(-1)
    cand_v, cand_i = _make_select_call()(dists)
    return _merge_call(cand_v.reshape(N_TILES, K_NB),
                       cand_i.reshape(N_TILES, K_NB))


# speedup vs baseline: 4.7863x; 1.2494x over previous
import functools

import jax
import jax.numpy as jnp
from jax import lax
from jax.experimental import pallas as pl
from jax.experimental.pallas import tpu as pltpu
from jax.experimental.pallas import tpu_sc as plsc

K_NB = 128
D = 64
N_KEYS = 1_000_000
N_TILES = 32
LANES = 16

N_T = ((N_KEYS + N_TILES - 1) // N_TILES + 127) // 128 * 128
N_PAD = N_T * N_TILES
NV = N_T // LANES

B1 = 12544 if N_PAD % 12544 == 0 else N_T
G1 = N_PAD // B1


def _dist_kernel(q_ref, k_ref, o_ref):
    qv = q_ref[...]
    kb = k_ref[...]
    kq = lax.dot_general(qv, kb, (((1,), (0,)), ((), ())))
    ksq = jnp.sum(kb * kb, axis=0, keepdims=True)
    qsq = jnp.sum(qv * qv)
    d2 = (qsq + ksq) - 2.0 * kq
    d2 = jnp.maximum(d2, 0.0)
    dist = jnp.sqrt(d2 + 1e-12)
    bits = pltpu.bitcast(dist, jnp.int32)
    i = pl.program_id(0)
    gid = i * B1 + lax.broadcasted_iota(jnp.int32, (1, B1), 1)
    o_ref[...] = jnp.where(gid < N_KEYS, bits, jnp.int32(0x7F800000))[None]


_dist_call = pl.pallas_call(
    _dist_kernel,
    grid=(G1,),
    in_specs=[
        pl.BlockSpec((1, D), lambda i: (0, 0)),
        pl.BlockSpec((D, B1), lambda i: (0, i)),
    ],
    out_specs=pl.BlockSpec((1, 1, B1), lambda i: (i, 0, 0)),
    out_shape=jax.ShapeDtypeStruct((G1, 1, B1), jnp.int32),
)


def _select_body(dists, out_v, out_i, vals, hist, racc, ltv, lti, eqi, ov, oi):
    c = lax.axis_index("c")
    s = lax.axis_index("s")
    wid = s * 2 + c
    base = wid * N_T
    pltpu.sync_copy(dists.at[pl.ds(base, N_T)], vals)

    lane = lax.iota(jnp.int32, LANES)
    ones = jnp.ones((LANES,), jnp.int32)
    zeros16 = jnp.zeros((LANES,), jnp.int32)

    prefix = jnp.int32(0)
    k_rem = jnp.int32(K_NB)
    UN = 8
    for r in range(4):
        sh_d = 24 - 8 * r

        def zbody(j):
            hist[pl.ds(j * LANES, LANES)] = zeros16

        plsc.parallel_loop(0, 256, 1, unroll=8)(zbody)

        if r == 0:
            def hbody(i):
                v = vals[pl.ds(i * LANES, LANES)]
                byte = (v >> sh_d) & 255
                plsc.addupdate_scatter(hist, [byte * LANES + lane], ones)
        else:
            sh_hi = 32 - 8 * r
            pref_hi = prefix >> sh_hi

            def hbody(i):
                v = vals[pl.ds(i * LANES, LANES)]
                byte = (v >> sh_d) & 255
                ok = (v >> sh_hi) == pref_hi
                plsc.addupdate_scatter(hist, [byte * LANES + lane], ones,
                                       mask=ok)

        plsc.parallel_loop(0, NV, 1, unroll=UN)(hbody)

        def cbody(jo, acc):
            for u in range(4):
                acc = acc + hist[pl.ds((jo * 4 + u) * LANES, LANES)]
                racc[pl.ds((jo * 4 + u) * LANES, LANES)] = acc
            return acc

        lax.fori_loop(0, 256 // 4, cbody, zeros16)

        def sbody(_, lohi):
            lo, hi = lohi
            mid = (lo + hi) // 2
            sm = jnp.sum(racc[pl.ds(mid * LANES, LANES)])
            ok = sm >= k_rem
            return jnp.where(ok, lo, mid + 1), jnp.where(ok, mid, hi)

        digit, _ = lax.fori_loop(0, 8, sbody, (jnp.int32(0), jnp.int32(255)))
        pm1 = jnp.maximum(digit - 1, 0)
        cum_before = jnp.where(
            digit > 0, jnp.sum(racc[pl.ds(pm1 * LANES, LANES)]), jnp.int32(0))
        k_rem = k_rem - cum_before
        prefix = prefix | (digit << sh_d)

    v_bits = prefix

    def pbody(io, ptrs):
        p_lt, p_eq = ptrs
        for u in range(UN):
            i = io * UN + u
            v = vals[pl.ds(i * LANES, LANES)]
            lt = v < v_bits
            eq = v == v_bits
            gidx = base + i * LANES + lane
            plsc.store_compressed(ltv.at[pl.ds(p_lt, LANES)], v, mask=lt)
            plsc.store_compressed(lti.at[pl.ds(p_lt, LANES)], gidx, mask=lt)
            plsc.store_compressed(eqi.at[pl.ds(p_eq, LANES)], gidx, mask=eq)
            p_lt = p_lt + plsc.all_reduce_population_count(lt)[0]
            p_eq = p_eq + plsc.all_reduce_population_count(eq)[0]
        return (p_lt, p_eq)

    lax.fori_loop(0, NV // UN, pbody, (jnp.int32(0), jnp.int32(0)))

    count_lt = jnp.int32(K_NB) - k_rem
    vfull = jnp.broadcast_to(v_bits, (LANES,))
    for j in range(K_NB // LANES):
        pos = j * LANES + lane
        sel = pos < count_lt
        lv = ltv[pl.ds(j * LANES, LANES)]
        li = lti[pl.ds(j * LANES, LANES)]
        ei = plsc.load_gather(eqi, [jnp.maximum(pos - count_lt, 0)])
        ov[pl.ds(j * LANES, LANES)] = jnp.where(sel, lv, vfull)
        oi[pl.ds(j * LANES, LANES)] = jnp.where(sel, li, ei)

    pltpu.sync_copy(ov, out_v.at[pl.ds(wid * K_NB, K_NB)])
    pltpu.sync_copy(oi, out_i.at[pl.ds(wid * K_NB, K_NB)])


@functools.lru_cache(maxsize=1)
def _make_select_call():
  return pl.kernel(
    _select_body,
    out_type=(
        jax.ShapeDtypeStruct((N_TILES * K_NB,), jnp.int32),
        jax.ShapeDtypeStruct((N_TILES * K_NB,), jnp.int32),
    ),
    mesh=plsc.VectorSubcoreMesh(
        core_axis_name="c", subcore_axis_name="s",
        num_cores=2, num_subcores=16),
    compiler_params=pltpu.CompilerParams(needs_layout_passes=False),
    scratch_types=[
        pltpu.VMEM((N_T,), jnp.int32),
        pltpu.VMEM((256 * LANES,), jnp.int32),
        pltpu.VMEM((256 * LANES,), jnp.int32),
        pltpu.VMEM((K_NB + LANES,), jnp.int32),
        pltpu.VMEM((K_NB + LANES,), jnp.int32),
        pltpu.VMEM((N_T + LANES,), jnp.int32),
        pltpu.VMEM((K_NB,), jnp.int32),
        pltpu.VMEM((K_NB,), jnp.int32),
    ],
  )


def _merge_kernel(cv_ref, ci_ref, ov_ref, oi_ref):
    lane = lax.broadcasted_iota(jnp.int32, (1, K_NB), 1)
    cand = pltpu.bitcast(cv_ref[...], jnp.float32)

    def body(j, carry):
        vals, oval, oidx = carry
        m = jnp.min(vals)
        sel = vals == m
        i = jnp.min(jnp.where(sel, ci_ref[...], jnp.int32(2**31 - 1)))
        vals = jnp.where(sel & (ci_ref[...] == i), jnp.float32(jnp.inf), vals)
        oval = jnp.where(lane == j, m, oval)
        oidx = jnp.where(lane == j, i, oidx)
        return vals, oval, oidx

    _, oval, oidx = lax.fori_loop(
        0, K_NB, body,
        (cand, jnp.zeros((1, K_NB), jnp.float32),
         jnp.zeros((1, K_NB), jnp.int32)))
    ov_ref[...] = oval
    oi_ref[...] = oidx


_merge_call = pl.pallas_call(
    _merge_kernel,
    out_shape=(
        jax.ShapeDtypeStruct((1, K_NB), jnp.float32),
        jax.ShapeDtypeStruct((1, K_NB), jnp.int32),
    ),
)


def kernel(queries, keys):
    dists = _dist_call(queries, keys.T).reshape(-1)
    cand_v, cand_i = _make_select_call()(dists)
    return _merge_call(cand_v.reshape(N_TILES, K_NB),
                       cand_i.reshape(N_TILES, K_NB))


# speedup vs baseline: 5.3669x; 1.1213x over previous
import functools

import jax
import jax.numpy as jnp
from jax import lax
from jax.experimental import pallas as pl
from jax.experimental.pallas import tpu as pltpu
from jax.experimental.pallas import tpu_sc as plsc

K_NB = 128
D = 64
N_KEYS = 1_000_000
N_TILES = 32
LANES = 16

N_T = ((N_KEYS + N_TILES - 1) // N_TILES + 127) // 128 * 128
N_PAD = N_T * N_TILES
NV = N_T // LANES

B1 = 25088 if N_PAD % 25088 == 0 else N_T
G1 = N_PAD // B1


def _dist_kernel(q_ref, k_ref, o_ref):
    qv = q_ref[...]
    kb = k_ref[...]
    kq = lax.dot_general(qv, kb, (((1,), (0,)), ((), ())))
    ksq = jnp.sum(kb * kb, axis=0, keepdims=True)
    qsq = jnp.sum(qv * qv)
    d2 = (qsq + ksq) - 2.0 * kq
    d2 = jnp.maximum(d2, 0.0)
    dist = jnp.sqrt(d2 + 1e-12)
    bits = pltpu.bitcast(dist, jnp.int32)
    i = pl.program_id(0)
    gid = i * B1 + lax.broadcasted_iota(jnp.int32, (1, B1), 1)
    o_ref[...] = jnp.where(gid < N_KEYS, bits, jnp.int32(0x7F800000))[None]


_dist_call = pl.pallas_call(
    _dist_kernel,
    grid=(G1,),
    in_specs=[
        pl.BlockSpec((1, D), lambda i: (0, 0)),
        pl.BlockSpec((D, B1), lambda i: (0, i)),
    ],
    out_specs=pl.BlockSpec((1, 1, B1), lambda i: (i, 0, 0)),
    out_shape=jax.ShapeDtypeStruct((G1, 1, B1), jnp.int32),
)


def _select_body(dists, out_v, out_i, vals, hist, racc, ltv, lti, eqi, ov, oi):
    c = lax.axis_index("c")
    s = lax.axis_index("s")
    wid = s * 2 + c
    base = wid * N_T
    pltpu.sync_copy(dists.at[pl.ds(base, N_T)], vals)

    lane = lax.iota(jnp.int32, LANES)
    ones = jnp.ones((LANES,), jnp.int32)
    zeros16 = jnp.zeros((LANES,), jnp.int32)

    prefix = jnp.int32(0)
    k_rem = jnp.int32(K_NB)
    UN = 8
    for r in range(4):
        sh_d = 24 - 8 * r

        def zbody(j):
            hist[pl.ds(j * LANES, LANES)] = zeros16

        plsc.parallel_loop(0, 256, 1, unroll=8)(zbody)

        if r == 0:
            def hbody(i):
                v = vals[pl.ds(i * LANES, LANES)]
                byte = (v >> sh_d) & 255
                plsc.addupdate_scatter(hist, [byte * LANES + lane], ones)
        else:
            sh_hi = 32 - 8 * r
            pref_hi = prefix >> sh_hi

            def hbody(i):
                v = vals[pl.ds(i * LANES, LANES)]
                byte = (v >> sh_d) & 255
                ok = (v >> sh_hi) == pref_hi
                plsc.addupdate_scatter(hist, [byte * LANES + lane], ones,
                                       mask=ok)

        plsc.parallel_loop(0, NV, 1, unroll=UN)(hbody)

        def cbody(jo, acc):
            for u in range(4):
                acc = acc + hist[pl.ds((jo * 4 + u) * LANES, LANES)]
                racc[pl.ds((jo * 4 + u) * LANES, LANES)] = acc
            return acc

        lax.fori_loop(0, 256 // 4, cbody, zeros16)

        def sbody(_, lohi):
            lo, hi = lohi
            mid = (lo + hi) // 2
            sm = jnp.sum(racc[pl.ds(mid * LANES, LANES)])
            ok = sm >= k_rem
            return jnp.where(ok, lo, mid + 1), jnp.where(ok, mid, hi)

        digit, _ = lax.fori_loop(0, 8, sbody, (jnp.int32(0), jnp.int32(255)))
        pm1 = jnp.maximum(digit - 1, 0)
        cum_before = jnp.where(
            digit > 0, jnp.sum(racc[pl.ds(pm1 * LANES, LANES)]), jnp.int32(0))
        k_rem = k_rem - cum_before
        prefix = prefix | (digit << sh_d)

    v_bits = prefix

    def pbody(io, ptrs):
        p_lt, p_eq = ptrs
        for u in range(UN):
            i = io * UN + u
            v = vals[pl.ds(i * LANES, LANES)]
            lt = v < v_bits
            eq = v == v_bits
            gidx = base + i * LANES + lane
            plsc.store_compressed(ltv.at[pl.ds(p_lt, LANES)], v, mask=lt)
            plsc.store_compressed(lti.at[pl.ds(p_lt, LANES)], gidx, mask=lt)
            plsc.store_compressed(eqi.at[pl.ds(p_eq, LANES)], gidx, mask=eq)
            p_lt = p_lt + plsc.all_reduce_population_count(lt)[0]
            p_eq = p_eq + plsc.all_reduce_population_count(eq)[0]
        return (p_lt, p_eq)

    lax.fori_loop(0, NV // UN, pbody, (jnp.int32(0), jnp.int32(0)))

    count_lt = jnp.int32(K_NB) - k_rem
    vfull = jnp.broadcast_to(v_bits, (LANES,))
    for j in range(K_NB // LANES):
        pos = j * LANES + lane
        sel = pos < count_lt
        lv = ltv[pl.ds(j * LANES, LANES)]
        li = lti[pl.ds(j * LANES, LANES)]
        ei = plsc.load_gather(eqi, [jnp.maximum(pos - count_lt, 0)])
        ov[pl.ds(j * LANES, LANES)] = jnp.where(sel, lv, vfull)
        oi[pl.ds(j * LANES, LANES)] = jnp.where(sel, li, ei)

    pltpu.sync_copy(ov, out_v.at[wid])
    pltpu.sync_copy(oi, out_i.at[wid])


@functools.lru_cache(maxsize=1)
def _make_select_call():
  return pl.kernel(
    _select_body,
    out_type=(
        jax.ShapeDtypeStruct((N_TILES, K_NB), jnp.int32),
        jax.ShapeDtypeStruct((N_TILES, K_NB), jnp.int32),
    ),
    mesh=plsc.VectorSubcoreMesh(
        core_axis_name="c", subcore_axis_name="s",
        num_cores=2, num_subcores=16),
    compiler_params=pltpu.CompilerParams(needs_layout_passes=False),
    scratch_types=[
        pltpu.VMEM((N_T,), jnp.int32),
        pltpu.VMEM((256 * LANES,), jnp.int32),
        pltpu.VMEM((256 * LANES,), jnp.int32),
        pltpu.VMEM((K_NB + LANES,), jnp.int32),
        pltpu.VMEM((K_NB + LANES,), jnp.int32),
        pltpu.VMEM((N_T + LANES,), jnp.int32),
        pltpu.VMEM((K_NB,), jnp.int32),
        pltpu.VMEM((K_NB,), jnp.int32),
    ],
  )


def _merge_kernel(cv_ref, ci_ref, ov_ref, oi_ref):
    lane = lax.broadcasted_iota(jnp.int32, (1, K_NB), 1)
    cand = pltpu.bitcast(cv_ref[...], jnp.float32)

    def body(j, carry):
        vals, oval, oidx = carry
        m = jnp.min(vals)
        sel = vals == m
        i = jnp.min(jnp.where(sel, ci_ref[...], jnp.int32(2**31 - 1)))
        vals = jnp.where(sel & (ci_ref[...] == i), jnp.float32(jnp.inf), vals)
        oval = jnp.where(lane == j, m, oval)
        oidx = jnp.where(lane == j, i, oidx)
        return vals, oval, oidx

    _, oval, oidx = lax.fori_loop(
        0, K_NB, body,
        (cand, jnp.zeros((1, K_NB), jnp.float32),
         jnp.zeros((1, K_NB), jnp.int32)))
    ov_ref[...] = oval
    oi_ref[...] = oidx


_merge_call = pl.pallas_call(
    _merge_kernel,
    out_shape=(
        jax.ShapeDtypeStruct((1, K_NB), jnp.float32),
        jax.ShapeDtypeStruct((1, K_NB), jnp.int32),
    ),
)


def kernel(queries, keys):
    dists = _dist_call(queries, keys.T).reshape(-1)
    cand_v, cand_i = _make_select_call()(dists)
    return _merge_call(cand_v, cand_i)
